# Initial kernel scaffold; baseline (speedup 1.0000x reference)
import jax
import jax.numpy as jnp
from jax.experimental import pallas as pl


def kernel(x, edge_index, edge_attr, batch, params):
    raise NotImplementedError("write your pallas kernel here")


# speedup vs baseline: 1.0000x; 1.0000x over previous
import jax, jax.numpy as jnp
from jax.experimental import pallas as pl

N = 10240
E = 655360
G = 40
NPG = 256
D = 16
H = 4
HC = H * D
EPS = 1e-5


def _segment_softmax(alpha, seg, num_segments):
    m = jax.ops.segment_max(alpha, seg, num_segments=num_segments)
    m = jnp.where(jnp.isfinite(m), m, 0.0)
    a = jnp.exp(alpha - m[seg])
    s = jax.ops.segment_sum(a, seg, num_segments=num_segments)
    return a / (s[seg] + 1e-16)


def _transformer_conv(x, edge_index, edge_attr, p):
    src = edge_index[0]; dst = edge_index[1]
    q = (x @ p['Wq'] + p['bq']).reshape(-1, H, D)
    k = (x @ p['Wk'] + p['bk']).reshape(-1, H, D)
    v = (x @ p['Wv'] + p['bv']).reshape(-1, H, D)
    e = (edge_attr @ p['We'] + p['be']).reshape(-1, H, D)
    kj = k[src] + e
    qi = q[dst]
    alpha = (qi * kj).sum(-1) / jnp.sqrt(float(D))
    alpha = _segment_softmax(alpha, dst, x.shape[0])
    msg = (v[src] + e) * alpha[..., None]
    out = jax.ops.segment_sum(msg, dst, num_segments=x.shape[0]).reshape(-1, HC)
    xr = x @ p['Wskip'] + p['bskip']
    beta = jax.nn.sigmoid(jnp.concatenate([out, xr, out - xr], axis=-1) @ p['Wbeta'])
    return beta * xr + (1.0 - beta) * out


def _graph_norm(x, batch, p, num_graphs):
    ones = jnp.ones((x.shape[0], 1), x.dtype)
    cnt = jax.ops.segment_sum(ones, batch, num_segments=num_graphs)
    mean = jax.ops.segment_sum(x, batch, num_segments=num_graphs) / cnt
    out = x - p['gn_alpha'] * mean[batch]
    var = jax.ops.segment_sum(out * out, batch, num_segments=num_graphs) / cnt
    std = jnp.sqrt(var + EPS)
    return p['gn_weight'] * out / std[batch] + p['gn_bias']


def _block(x, edge_index, edge_attr, batch, p):
    h = _transformer_conv(x, edge_index, edge_attr, p)
    h = jax.nn.relu(h @ p['Wlin'] + p['blin'])
    return _graph_norm(h, batch, p, G)


def _conv2d(x, w, b, padding):
    y = jax.lax.conv_general_dilated(x, w, (1, 1), padding, dimension_numbers=('NCHW', 'OIHW', 'NCHW'))
    return y + b[None, :, None, None]


def _contact_cnn(z0, z1, cnn):
    z_dif = jnp.abs(z0[:, :, None, :] - z1[:, None, :, :])
    z_mul = z0[:, :, None, :] * z1[:, None, :, :]
    zc = jnp.concatenate([z_dif, z_mul], axis=-1)
    h = jnp.transpose(zc, (0, 3, 1, 2))
    h = jax.nn.relu(_conv2d(h, cnn['W0'], cnn['b0'], 'SAME'))
    for w, b in zip(cnn['Wres'], cnn['bres']):
        h = jax.nn.relu(h + jax.nn.relu(_conv2d(h, w, b, 'SAME')))
    c = jax.nn.sigmoid(_conv2d(h, cnn['Wf'], cnn['bf'], 'SAME'))
    c = 0.5 * (c + jnp.transpose(c, (0, 1, 3, 2)))
    return c[:, 0, :, :]


def kernel(x, edge_index, edge_attr, batch, params):
    h = x
    for p in params['blocks']:
        h = _block(h, edge_index, edge_attr, batch, p)
    Z = h.reshape(G, NPG, D)
    return _contact_cnn(Z, Z, params['cnn'])


# speedup vs baseline: 48.8810x; 48.8807x over previous
import functools

import jax
import jax.numpy as jnp
from jax import lax
from jax.experimental import pallas as pl
from jax.experimental.pallas import tpu as pltpu
from jax.experimental.pallas import tpu_sc as plsc

N = 10240
E = 655360
G = 40
NPG = 256
D = 16
H = 4
HC = H * D
EPS = 1e-5

NW = 32
CHUNK = 128
EPW = E // NW
PW = 128
RPS = N // 16

def _sc_mesh():
    return plsc.VectorSubcoreMesh(core_axis_name="c", subcore_axis_name="s")


def _proj_body(x_ref, wq_ref, wkv_ref, ws_ref, bq_ref, bkv_ref, bs_ref,
               q_ref, kv_ref, xr_ref):
    x = x_ref[...]
    qm = jnp.dot(x, wq_ref[...], preferred_element_type=jnp.float32) + bq_ref[...]
    q_ref[...] = jnp.concatenate(
        [qm, jnp.zeros((qm.shape[0], 2 * HC - HC), jnp.float32)], axis=1)
    kv_ref[...] = jnp.dot(x, wkv_ref[...],
                          preferred_element_type=jnp.float32) + bkv_ref[...]
    xr_ref[...] = jnp.dot(x, ws_ref[...],
                          preferred_element_type=jnp.float32) + bs_ref[...]


def _proj(x, p):
    din = x.shape[1]
    wq = p['Wq']
    wkv = jnp.concatenate([p['Wk'], p['Wv']], axis=1)
    ws = p['Wskip']
    bq = p['bq'].reshape(1, HC)
    bkv = jnp.concatenate([p['bk'], p['bv']]).reshape(1, 2 * HC)
    bs = p['bskip'].reshape(1, HC)
    bn = 1024
    grid = (N // bn,)
    return pl.pallas_call(
        _proj_body,
        grid=grid,
        in_specs=[
            pl.BlockSpec((bn, din), lambda i: (i, 0)),
            pl.BlockSpec((din, HC), lambda i: (0, 0)),
            pl.BlockSpec((din, 2 * HC), lambda i: (0, 0)),
            pl.BlockSpec((din, HC), lambda i: (0, 0)),
            pl.BlockSpec((1, HC), lambda i: (0, 0)),
            pl.BlockSpec((1, 2 * HC), lambda i: (0, 0)),
            pl.BlockSpec((1, HC), lambda i: (0, 0)),
        ],
        out_specs=[
            pl.BlockSpec((bn, 2 * HC), lambda i: (i, 0)),
            pl.BlockSpec((bn, 2 * HC), lambda i: (i, 0)),
            pl.BlockSpec((bn, HC), lambda i: (i, 0)),
        ],
        out_shape=[
            jax.ShapeDtypeStruct((N, 2 * HC), jnp.float32),
            jax.ShapeDtypeStruct((N, 2 * HC), jnp.float32),
            jax.ShapeDtypeStruct((N, HC), jnp.float32),
        ],
    )(x, wq, wkv, ws, bq, bkv, bs)


def _gather_body(q_hbm, kv_hbm, dst_hbm, src_hbm, qd_out, kv_out,
                 idx_d, idx_s, qbuf, kvbuf, sem0, sem1):
    c = lax.axis_index("c")
    s = lax.axis_index("s")
    wid = s * 2 + c
    base0 = wid * EPW

    def chunk(i, carry):
        b = base0 + i * CHUNK
        pltpu.sync_copy(dst_hbm.at[pl.ds(b, CHUNK)], idx_d)
        pltpu.sync_copy(src_hbm.at[pl.ds(b, CHUNK)], idx_s)
        cp0 = pltpu.async_copy(q_hbm.at[idx_d], qbuf, sem0)
        cp1 = pltpu.async_copy(kv_hbm.at[idx_s], kvbuf, sem1)
        cp0.wait()
        cp1.wait()
        pltpu.sync_copy(qbuf, qd_out.at[pl.ds(b, CHUNK)])
        pltpu.sync_copy(kvbuf, kv_out.at[pl.ds(b, CHUNK)])
        return carry

    lax.fori_loop(0, EPW // CHUNK, chunk, 0)


def _gather(q, kv, dst, src):
    k = pl.kernel(
        _gather_body,
        out_type=[
            jax.ShapeDtypeStruct((E, 2 * HC), jnp.float32),
            jax.ShapeDtypeStruct((E, 2 * HC), jnp.float32),
        ],
        mesh=_sc_mesh(),
        scratch_types=[
            pltpu.VMEM((CHUNK,), jnp.int32),
            pltpu.VMEM((CHUNK,), jnp.int32),
            pltpu.VMEM((CHUNK, 2 * HC), jnp.float32),
            pltpu.VMEM((CHUNK, 2 * HC), jnp.float32),
            pltpu.SemaphoreType.DMA,
            pltpu.SemaphoreType.DMA,
        ],
    )
    return k(q, kv, dst, src)


def _edge_body(qd_ref, kv_ref, a_ref, hm_ref, hmt_ref, web_ref, pay_ref):
    qd = qd_ref[:, :HC]
    kvb = kv_ref[...]
    kk = kvb[:, :HC]
    vv = kvb[:, HC:]
    e = a_ref[...] * web_ref[0:1, :] + web_ref[1:2, :]
    logit = jnp.dot(qd * (kk + e), hm_ref[...],
                    preferred_element_type=jnp.float32) * 0.25
    w4 = jnp.exp(logit)
    w64 = jnp.dot(w4, hmt_ref[...], preferred_element_type=jnp.float32)
    msg = w64 * (vv + e)
    pad = jnp.zeros((msg.shape[0], PW - HC - H), jnp.float32)
    pay_ref[...] = jnp.concatenate([msg, w4, pad], axis=1)


def _edges(qd, kvs, a, web, hm, hmt):
    bn = 4096
    grid = (E // bn,)
    return pl.pallas_call(
        _edge_body,
        grid=grid,
        in_specs=[
            pl.BlockSpec((bn, 2 * HC), lambda i: (i, 0)),
            pl.BlockSpec((bn, 2 * HC), lambda i: (i, 0)),
            pl.BlockSpec((bn, 1), lambda i: (i, 0)),
            pl.BlockSpec((HC, H), lambda i: (0, 0)),
            pl.BlockSpec((H, HC), lambda i: (0, 0)),
            pl.BlockSpec((2, HC), lambda i: (0, 0)),
        ],
        out_specs=pl.BlockSpec((bn, PW), lambda i: (i, 0)),
        out_shape=jax.ShapeDtypeStruct((E, PW), jnp.float32),
    )(qd, kvs, a, hm, hmt, web)


def _scatter_body(pay_hbm, dst_hbm, zeros_hbm, out_hbm, idx_v, pbuf, acc, sem):
    c = lax.axis_index("c")
    s = lax.axis_index("s")
    wid = s * 2 + c
    base0 = wid * EPW
    pltpu.sync_copy(zeros_hbm.at[pl.ds(s * RPS, RPS)], acc.at[pl.ds(s * RPS, RPS)])
    plsc.subcore_barrier()

    def chunk(i, carry):
        b = base0 + i * CHUNK
        pltpu.sync_copy(dst_hbm.at[pl.ds(b, CHUNK)], idx_v)
        pltpu.sync_copy(pay_hbm.at[pl.ds(b, CHUNK)], pbuf)
        pltpu.sync_copy(pbuf, acc.at[idx_v], add=True)
        return carry

    lax.fori_loop(0, EPW // CHUNK, chunk, 0)
    plsc.subcore_barrier()
    pltpu.sync_copy(acc.at[pl.ds(s * RPS, RPS)], out_hbm.at[c, pl.ds(s * RPS, RPS)])


def _scatter(pay, dst, zeros):
    k = pl.kernel(
        _scatter_body,
        out_type=jax.ShapeDtypeStruct((2, N, PW), jnp.float32),
        mesh=_sc_mesh(),
        scratch_types=[
            pltpu.VMEM((CHUNK,), jnp.int32),
            pltpu.VMEM((CHUNK, PW), jnp.float32),
            pltpu.VMEM_SHARED((N, PW), jnp.float32),
            pltpu.SemaphoreType.DMA,
        ],
    )
    return k(pay, dst, zeros)


def _combine_body(parts_ref, xr_ref, hmt_ref, wb_ref, wlin_ref, blin_ref,
                  gnw_ref, gnb_ref, gna_ref, h_ref):
    p = parts_ref[0] + parts_ref[1]
    u = p[:, :HC]
    s4 = p[:, HC:HC + H]
    s64 = jnp.dot(s4, hmt_ref[...], preferred_element_type=jnp.float32)
    out = u / (s64 + 1e-16)
    xr = xr_ref[...]
    cat = jnp.concatenate([out, xr, out - xr], axis=1)
    beta = jax.nn.sigmoid(jnp.dot(cat, wb_ref[...],
                                  preferred_element_type=jnp.float32))
    h1 = beta * xr + (1.0 - beta) * out
    h2 = jax.nn.relu(jnp.dot(h1, wlin_ref[...],
                             preferred_element_type=jnp.float32) + blin_ref[...])
    mean = jnp.mean(h2, axis=0, keepdims=True)
    og = h2 - gna_ref[...] * mean
    var = jnp.mean(og * og, axis=0, keepdims=True)
    h_ref[...] = gnw_ref[...] * og / jnp.sqrt(var + EPS) + gnb_ref[...]


def _combine(parts, xr, p, hmt):
    grid = (G,)
    return pl.pallas_call(
        _combine_body,
        grid=grid,
        in_specs=[
            pl.BlockSpec((2, NPG, PW), lambda g: (0, g, 0)),
            pl.BlockSpec((NPG, HC), lambda g: (g, 0)),
            pl.BlockSpec((H, HC), lambda g: (0, 0)),
            pl.BlockSpec((3 * HC, 1), lambda g: (0, 0)),
            pl.BlockSpec((HC, D), lambda g: (0, 0)),
            pl.BlockSpec((1, D), lambda g: (0, 0)),
            pl.BlockSpec((1, D), lambda g: (0, 0)),
            pl.BlockSpec((1, D), lambda g: (0, 0)),
            pl.BlockSpec((1, D), lambda g: (0, 0)),
        ],
        out_specs=pl.BlockSpec((NPG, D), lambda g: (g, 0)),
        out_shape=jax.ShapeDtypeStruct((N, D), jnp.float32),
    )(parts, xr, hmt, p['Wbeta'], p['Wlin'], p['blin'].reshape(1, D),
      p['gn_weight'].reshape(1, D), p['gn_bias'].reshape(1, D),
      p['gn_alpha'].reshape(1, D))


_ST = 32
_NPIX = NPG * NPG


def _cnn_body(zt_ref, w0a_ref, w0b_ref, b0_ref, wr_ref, br_ref, wf_ref, bf_ref,
              out_ref, hp_ref, acc_ref, c_ref):
    z2 = zt_ref[0]
    hp_ref[:, 0:1, :] = jnp.zeros((D, 1, NPG + 2), jnp.float32)
    hp_ref[:, NPG + 1:NPG + 2, :] = jnp.zeros((D, 1, NPG + 2), jnp.float32)
    hp_ref[:, :, 0:1] = jnp.zeros((D, NPG + 2, 1), jnp.float32)
    hp_ref[:, :, NPG + 1:NPG + 2] = jnp.zeros((D, NPG + 2, 1), jnp.float32)

    w0a = w0a_ref[...]
    w0b = w0b_ref[...]
    b0c = b0_ref[...]
    for t in range(NPG // _ST):
        zi = z2[:, t * _ST:(t + 1) * _ST][:, :, None]
        zj = z2[:, None, :]
        dif = jnp.abs(zi - zj).reshape(D, _ST * NPG)
        mul = (zi * zj).reshape(D, _ST * NPG)
        h0s = jax.nn.relu(
            jnp.dot(w0a, dif, preferred_element_type=jnp.float32)
            + jnp.dot(w0b, mul, preferred_element_type=jnp.float32) + b0c)
        hp_ref[:, 1 + t * _ST:1 + (t + 1) * _ST, 1:NPG + 1] = (
            h0s.reshape(D, _ST, NPG))

    for di in range(3):
        for dj in range(3):
            sl = hp_ref[:, di:di + NPG, dj:dj + NPG].reshape(D, _NPIX)
            contrib = jnp.dot(wr_ref[di, dj], sl,
                              preferred_element_type=jnp.float32)
            if di == 0 and dj == 0:
                acc_ref[...] = contrib
            else:
                acc_ref[...] = acc_ref[...] + contrib

    brc = br_ref[...]
    wf = wf_ref[...]
    for t in range(NPG // _ST):
        h0s = hp_ref[:, 1 + t * _ST:1 + (t + 1) * _ST, 1:NPG + 1].reshape(
            D, _ST * NPG)
        rs = jax.nn.relu(acc_ref[:, t * _ST * NPG:(t + 1) * _ST * NPG] + brc)
        h1 = jax.nn.relu(h0s + rs)
        logit = jnp.dot(wf, h1, preferred_element_type=jnp.float32) + bf_ref[0, 0]
        c_ref[t * _ST:(t + 1) * _ST, :] = jax.nn.sigmoid(logit).reshape(_ST, NPG)

    cmap = c_ref[...]
    out_ref[0] = 0.5 * (cmap + cmap.T)


def _cnn(zt, cnn):
    w0 = cnn['W0'][:, :, 0, 0]
    w0a = w0[:, :D]
    w0b = w0[:, D:]
    wr = jnp.transpose(cnn['Wres'][0], (2, 3, 0, 1))
    wf = cnn['Wf'][:, :, 0, 0]
    grid = (G,)
    return pl.pallas_call(
        _cnn_body,
        grid=grid,
        in_specs=[
            pl.BlockSpec((1, D, NPG), lambda g: (g, 0, 0)),
            pl.BlockSpec((D, D), lambda g: (0, 0)),
            pl.BlockSpec((D, D), lambda g: (0, 0)),
            pl.BlockSpec((D, 1), lambda g: (0, 0)),
            pl.BlockSpec((3, 3, D, D), lambda g: (0, 0, 0, 0)),
            pl.BlockSpec((D, 1), lambda g: (0, 0)),
            pl.BlockSpec((1, D), lambda g: (0, 0)),
            pl.BlockSpec((1, 1), lambda g: (0, 0)),
        ],
        out_specs=pl.BlockSpec((1, NPG, NPG), lambda g: (g, 0, 0)),
        out_shape=jax.ShapeDtypeStruct((G, NPG, NPG), jnp.float32),
        scratch_shapes=[
            pltpu.VMEM((D, NPG + 2, NPG + 2), jnp.float32),
            pltpu.VMEM((D, _NPIX), jnp.float32),
            pltpu.VMEM((NPG, NPG), jnp.float32),
        ],
    )(zt, w0a, w0b, cnn['b0'].reshape(D, 1), wr, cnn['bres'][0].reshape(D, 1),
      wf, cnn['bf'].reshape(1, 1))


def kernel(x, edge_index, edge_attr, batch, params):
    del batch
    dst = edge_index[1]
    src = edge_index[0]
    a = edge_attr
    hm = (jnp.arange(HC)[:, None] // D == jnp.arange(H)[None, :]).astype(jnp.float32)
    hmt = hm.T
    zeros = jnp.zeros((N, PW), jnp.float32)

    h = x
    for p in params['blocks']:
        q, kv, xr = _proj(h, p)
        qd, kvs = _gather(q, kv, dst, src)
        web = jnp.stack([p['We'][0], p['be']], axis=0)
        pay = _edges(qd, kvs, a, web, hm, hmt)
        parts = _scatter(pay, dst, zeros)
        h = _combine(parts, xr, p, hmt)

    zt = jnp.transpose(h.reshape(G, NPG, D), (0, 2, 1))
    return _cnn(zt, params['cnn'])


# speedup vs baseline: 53.7848x; 1.1003x over previous
import functools

import jax
import jax.numpy as jnp
from jax import lax
from jax.experimental import pallas as pl
from jax.experimental.pallas import tpu as pltpu
from jax.experimental.pallas import tpu_sc as plsc

N = 10240
E = 655360
G = 40
NPG = 256
D = 16
H = 4
HC = H * D
EPS = 1e-5

NW = 32
CHUNK = 128
EPW = E // NW
PW = 80
RPS = N // 16

def _sc_mesh():
    return plsc.VectorSubcoreMesh(core_axis_name="c", subcore_axis_name="s")


def _proj_body(x_ref, wq_ref, wkv_ref, ws_ref, bq_ref, bkv_ref, bs_ref,
               q_ref, kv_ref, xr_ref):
    x = x_ref[...]
    qm = jnp.dot(x, wq_ref[...], preferred_element_type=jnp.float32) + bq_ref[...]
    q_ref[...] = jnp.concatenate(
        [qm, jnp.zeros((qm.shape[0], 2 * HC - HC), jnp.float32)], axis=1)
    kv_ref[...] = jnp.dot(x, wkv_ref[...],
                          preferred_element_type=jnp.float32) + bkv_ref[...]
    xr_ref[...] = jnp.dot(x, ws_ref[...],
                          preferred_element_type=jnp.float32) + bs_ref[...]


def _proj(x, p):
    din = x.shape[1]
    wq = p['Wq']
    wkv = jnp.concatenate([p['Wk'], p['Wv']], axis=1)
    ws = p['Wskip']
    bq = p['bq'].reshape(1, HC)
    bkv = jnp.concatenate([p['bk'], p['bv']]).reshape(1, 2 * HC)
    bs = p['bskip'].reshape(1, HC)
    bn = 1024
    grid = (N // bn,)
    return pl.pallas_call(
        _proj_body,
        grid=grid,
        in_specs=[
            pl.BlockSpec((bn, din), lambda i: (i, 0)),
            pl.BlockSpec((din, HC), lambda i: (0, 0)),
            pl.BlockSpec((din, 2 * HC), lambda i: (0, 0)),
            pl.BlockSpec((din, HC), lambda i: (0, 0)),
            pl.BlockSpec((1, HC), lambda i: (0, 0)),
            pl.BlockSpec((1, 2 * HC), lambda i: (0, 0)),
            pl.BlockSpec((1, HC), lambda i: (0, 0)),
        ],
        out_specs=[
            pl.BlockSpec((bn, 2 * HC), lambda i: (i, 0)),
            pl.BlockSpec((bn, 2 * HC), lambda i: (i, 0)),
            pl.BlockSpec((bn, HC), lambda i: (i, 0)),
        ],
        out_shape=[
            jax.ShapeDtypeStruct((N, 2 * HC), jnp.float32),
            jax.ShapeDtypeStruct((N, 2 * HC), jnp.float32),
            jax.ShapeDtypeStruct((N, HC), jnp.float32),
        ],
    )(x, wq, wkv, ws, bq, bkv, bs)


_RPW = EPW // CHUNK


def _gather_body(q_hbm, kv_hbm, dstm_hbm, srcm_hbm, qd_out, kv_out,
                 id0, id1, is0, is1, qb0, qb1, kb0, kb1, semi, semg0, semg1,
                 semg2, semg3, semw):
    c = lax.axis_index("c")
    s = lax.axis_index("s")
    wid = s * 2 + c
    rbase = wid * _RPW

    def group(g, carry):
        r0 = rbase + 2 * g
        b0 = r0 * CHUNK
        i1 = pltpu.async_copy(dstm_hbm.at[r0], id0, semi)
        i2 = pltpu.async_copy(srcm_hbm.at[r0], is0, semi)
        i3 = pltpu.async_copy(dstm_hbm.at[r0 + 1], id1, semi)
        i4 = pltpu.async_copy(srcm_hbm.at[r0 + 1], is1, semi)
        i1.wait(); i2.wait(); i3.wait(); i4.wait()
        g1 = pltpu.async_copy(q_hbm.at[id0], qb0, semg0)
        g2 = pltpu.async_copy(kv_hbm.at[is0], kb0, semg1)
        g3 = pltpu.async_copy(q_hbm.at[id1], qb1, semg2)
        g4 = pltpu.async_copy(kv_hbm.at[is1], kb1, semg3)
        g1.wait(); g2.wait(); g3.wait(); g4.wait()
        w1 = pltpu.async_copy(qb0, qd_out.at[pl.ds(b0, CHUNK)], semw)
        w2 = pltpu.async_copy(kb0, kv_out.at[pl.ds(b0, CHUNK)], semw)
        w3 = pltpu.async_copy(qb1, qd_out.at[pl.ds(b0 + CHUNK, CHUNK)], semw)
        w4 = pltpu.async_copy(kb1, kv_out.at[pl.ds(b0 + CHUNK, CHUNK)], semw)
        w1.wait(); w2.wait(); w3.wait(); w4.wait()
        return carry

    lax.fori_loop(0, _RPW // 2, group, 0)


def _gather(q, kv, dstm, srcm):
    k = pl.kernel(
        _gather_body,
        out_type=[
            jax.ShapeDtypeStruct((E, 2 * HC), jnp.float32),
            jax.ShapeDtypeStruct((E, 2 * HC), jnp.float32),
        ],
        mesh=_sc_mesh(),
        scratch_types=[
            pltpu.VMEM((CHUNK,), jnp.int32),
            pltpu.VMEM((CHUNK,), jnp.int32),
            pltpu.VMEM((CHUNK,), jnp.int32),
            pltpu.VMEM((CHUNK,), jnp.int32),
            pltpu.VMEM((CHUNK, 2 * HC), jnp.float32),
            pltpu.VMEM((CHUNK, 2 * HC), jnp.float32),
            pltpu.VMEM((CHUNK, 2 * HC), jnp.float32),
            pltpu.VMEM((CHUNK, 2 * HC), jnp.float32),
            pltpu.SemaphoreType.DMA,
            pltpu.SemaphoreType.DMA,
            pltpu.SemaphoreType.DMA,
            pltpu.SemaphoreType.DMA,
            pltpu.SemaphoreType.DMA,
            pltpu.SemaphoreType.DMA,
        ],
    )
    return k(q, kv, dstm, srcm)


def _edge_body(qd_ref, kv_ref, a_ref, hm_ref, hmt_ref, web_ref, pay_ref):
    qd = qd_ref[:, :HC]
    kvb = kv_ref[...]
    kk = kvb[:, :HC]
    vv = kvb[:, HC:]
    e = a_ref[...] * web_ref[0:1, :] + web_ref[1:2, :]
    logit = jnp.dot(qd * (kk + e), hm_ref[...],
                    preferred_element_type=jnp.float32) * 0.25
    w4 = jnp.exp(logit)
    w64 = jnp.dot(w4, hmt_ref[...], preferred_element_type=jnp.float32)
    msg = w64 * (vv + e)
    pad = jnp.zeros((msg.shape[0], PW - HC - H), jnp.float32)
    pay_ref[...] = jnp.concatenate([msg, w4, pad], axis=1)


def _edges(qd, kvs, a, web, hm, hmt):
    bn = 4096
    grid = (E // bn,)
    return pl.pallas_call(
        _edge_body,
        grid=grid,
        in_specs=[
            pl.BlockSpec((bn, 2 * HC), lambda i: (i, 0)),
            pl.BlockSpec((bn, 2 * HC), lambda i: (i, 0)),
            pl.BlockSpec((bn, 1), lambda i: (i, 0)),
            pl.BlockSpec((HC, H), lambda i: (0, 0)),
            pl.BlockSpec((H, HC), lambda i: (0, 0)),
            pl.BlockSpec((2, HC), lambda i: (0, 0)),
        ],
        out_specs=pl.BlockSpec((bn, PW), lambda i: (i, 0)),
        out_shape=jax.ShapeDtypeStruct((E, PW), jnp.float32),
    )(qd, kvs, a, hm, hmt, web)


def _scatter_body(pay_hbm, dstm_hbm, zeros_hbm, out_hbm, idx_v, pbuf, acc, sem):
    c = lax.axis_index("c")
    s = lax.axis_index("s")
    wid = s * 2 + c
    rbase = wid * _RPW
    pltpu.sync_copy(zeros_hbm.at[pl.ds(s * RPS, RPS)], acc.at[pl.ds(s * RPS, RPS)])
    plsc.subcore_barrier()

    def chunk(i, carry):
        b = (rbase + i) * CHUNK
        pltpu.sync_copy(dstm_hbm.at[rbase + i], idx_v)
        pltpu.sync_copy(pay_hbm.at[pl.ds(b, CHUNK)], pbuf)
        pltpu.sync_copy(pbuf, acc.at[idx_v], add=True)
        return carry

    lax.fori_loop(0, _RPW, chunk, 0)
    plsc.subcore_barrier()
    pltpu.sync_copy(acc.at[pl.ds(s * RPS, RPS)], out_hbm.at[c, pl.ds(s * RPS, RPS)])


def _scatter(pay, dstm, zeros):
    k = pl.kernel(
        _scatter_body,
        out_type=jax.ShapeDtypeStruct((2, N, PW), jnp.float32),
        mesh=_sc_mesh(),
        scratch_types=[
            pltpu.VMEM((CHUNK,), jnp.int32),
            pltpu.VMEM((CHUNK, PW), jnp.float32),
            pltpu.VMEM_SHARED((N, PW), jnp.float32),
            pltpu.SemaphoreType.DMA,
        ],
    )
    return k(pay, dstm, zeros)


def _combine_body(parts_ref, xr_ref, hmt_ref, wb_ref, wlin_ref, blin_ref,
                  gnw_ref, gnb_ref, gna_ref, h_ref):
    p = parts_ref[0] + parts_ref[1]
    u = p[:, :HC]
    s4 = p[:, HC:HC + H]
    s64 = jnp.dot(s4, hmt_ref[...], preferred_element_type=jnp.float32)
    out = u / (s64 + 1e-16)
    xr = xr_ref[...]
    cat = jnp.concatenate([out, xr, out - xr], axis=1)
    beta = jax.nn.sigmoid(jnp.dot(cat, wb_ref[...],
                                  preferred_element_type=jnp.float32))
    h1 = beta * xr + (1.0 - beta) * out
    h2 = jax.nn.relu(jnp.dot(h1, wlin_ref[...],
                             preferred_element_type=jnp.float32) + blin_ref[...])
    mean = jnp.mean(h2, axis=0, keepdims=True)
    og = h2 - gna_ref[...] * mean
    var = jnp.mean(og * og, axis=0, keepdims=True)
    h_ref[...] = gnw_ref[...] * og / jnp.sqrt(var + EPS) + gnb_ref[...]


def _combine(parts, xr, p, hmt):
    grid = (G,)
    return pl.pallas_call(
        _combine_body,
        grid=grid,
        in_specs=[
            pl.BlockSpec((2, NPG, PW), lambda g: (0, g, 0)),
            pl.BlockSpec((NPG, HC), lambda g: (g, 0)),
            pl.BlockSpec((H, HC), lambda g: (0, 0)),
            pl.BlockSpec((3 * HC, 1), lambda g: (0, 0)),
            pl.BlockSpec((HC, D), lambda g: (0, 0)),
            pl.BlockSpec((1, D), lambda g: (0, 0)),
            pl.BlockSpec((1, D), lambda g: (0, 0)),
            pl.BlockSpec((1, D), lambda g: (0, 0)),
            pl.BlockSpec((1, D), lambda g: (0, 0)),
        ],
        out_specs=pl.BlockSpec((NPG, D), lambda g: (g, 0)),
        out_shape=jax.ShapeDtypeStruct((N, D), jnp.float32),
    )(parts, xr, hmt, p['Wbeta'], p['Wlin'], p['blin'].reshape(1, D),
      p['gn_weight'].reshape(1, D), p['gn_bias'].reshape(1, D),
      p['gn_alpha'].reshape(1, D))


_ST = 32
_NPIX = NPG * NPG


def _cnn_body(zt_ref, w0a_ref, w0b_ref, b0_ref, wr_ref, br_ref, wf_ref, bf_ref,
              out_ref, hp_ref, acc_ref, c_ref):
    z2 = zt_ref[0]
    hp_ref[:, 0:1, :] = jnp.zeros((D, 1, NPG + 2), jnp.float32)
    hp_ref[:, NPG + 1:NPG + 2, :] = jnp.zeros((D, 1, NPG + 2), jnp.float32)
    hp_ref[:, :, 0:1] = jnp.zeros((D, NPG + 2, 1), jnp.float32)
    hp_ref[:, :, NPG + 1:NPG + 2] = jnp.zeros((D, NPG + 2, 1), jnp.float32)

    w0a = w0a_ref[...]
    w0b = w0b_ref[...]
    b0c = b0_ref[...]
    for t in range(NPG // _ST):
        zi = z2[:, t * _ST:(t + 1) * _ST][:, :, None]
        zj = z2[:, None, :]
        dif = jnp.abs(zi - zj).reshape(D, _ST * NPG)
        mul = (zi * zj).reshape(D, _ST * NPG)
        h0s = jax.nn.relu(
            jnp.dot(w0a, dif, preferred_element_type=jnp.float32)
            + jnp.dot(w0b, mul, preferred_element_type=jnp.float32) + b0c)
        hp_ref[:, 1 + t * _ST:1 + (t + 1) * _ST, 1:NPG + 1] = (
            h0s.reshape(D, _ST, NPG))

    for di in range(3):
        for dj in range(3):
            sl = hp_ref[:, di:di + NPG, dj:dj + NPG].reshape(D, _NPIX)
            contrib = jnp.dot(wr_ref[di, dj], sl,
                              preferred_element_type=jnp.float32)
            if di == 0 and dj == 0:
                acc_ref[...] = contrib
            else:
                acc_ref[...] = acc_ref[...] + contrib

    brc = br_ref[...]
    wf = wf_ref[...]
    for t in range(NPG // _ST):
        h0s = hp_ref[:, 1 + t * _ST:1 + (t + 1) * _ST, 1:NPG + 1].reshape(
            D, _ST * NPG)
        rs = jax.nn.relu(acc_ref[:, t * _ST * NPG:(t + 1) * _ST * NPG] + brc)
        h1 = jax.nn.relu(h0s + rs)
        logit = jnp.dot(wf, h1, preferred_element_type=jnp.float32) + bf_ref[0, 0]
        c_ref[t * _ST:(t + 1) * _ST, :] = jax.nn.sigmoid(logit).reshape(_ST, NPG)

    cmap = c_ref[...]
    out_ref[0] = 0.5 * (cmap + cmap.T)


def _cnn(zt, cnn):
    w0 = cnn['W0'][:, :, 0, 0]
    w0a = w0[:, :D]
    w0b = w0[:, D:]
    wr = jnp.transpose(cnn['Wres'][0], (2, 3, 0, 1))
    wf = cnn['Wf'][:, :, 0, 0]
    grid = (G,)
    return pl.pallas_call(
        _cnn_body,
        grid=grid,
        in_specs=[
            pl.BlockSpec((1, D, NPG), lambda g: (g, 0, 0)),
            pl.BlockSpec((D, D), lambda g: (0, 0)),
            pl.BlockSpec((D, D), lambda g: (0, 0)),
            pl.BlockSpec((D, 1), lambda g: (0, 0)),
            pl.BlockSpec((3, 3, D, D), lambda g: (0, 0, 0, 0)),
            pl.BlockSpec((D, 1), lambda g: (0, 0)),
            pl.BlockSpec((1, D), lambda g: (0, 0)),
            pl.BlockSpec((1, 1), lambda g: (0, 0)),
        ],
        out_specs=pl.BlockSpec((1, NPG, NPG), lambda g: (g, 0, 0)),
        out_shape=jax.ShapeDtypeStruct((G, NPG, NPG), jnp.float32),
        scratch_shapes=[
            pltpu.VMEM((D, NPG + 2, NPG + 2), jnp.float32),
            pltpu.VMEM((D, _NPIX), jnp.float32),
            pltpu.VMEM((NPG, NPG), jnp.float32),
        ],
    )(zt, w0a, w0b, cnn['b0'].reshape(D, 1), wr, cnn['bres'][0].reshape(D, 1),
      wf, cnn['bf'].reshape(1, 1))


def kernel(x, edge_index, edge_attr, batch, params):
    del batch
    srcm = edge_index[0].reshape(E // CHUNK, CHUNK)
    dstm = edge_index[1].reshape(E // CHUNK, CHUNK)
    a = edge_attr
    hm = (jnp.arange(HC)[:, None] // D == jnp.arange(H)[None, :]).astype(jnp.float32)
    hmt = hm.T
    zeros = jnp.zeros((N, PW), jnp.float32)

    h = x
    for p in params['blocks']:
        q, kv, xr = _proj(h, p)
        qd, kvs = _gather(q, kv, dstm, srcm)
        web = jnp.stack([p['We'][0], p['be']], axis=0)
        pay = _edges(qd, kvs, a, web, hm, hmt)
        parts = _scatter(pay, dstm, zeros)
        h = _combine(parts, xr, p, hmt)

    zt = jnp.transpose(h.reshape(G, NPG, D), (0, 2, 1))
    return _cnn(zt, params['cnn'])
